# baseline (speedup 1.0000x reference)
import functools

import jax
import jax.numpy as jnp
from jax import lax
from jax.experimental import pallas as pl
from jax.experimental.pallas import tpu as pltpu
from jax.experimental.pallas import tpu_sc as plsc

NC = 2
NS = 16
GB = 64
NBUF = 4
SB = 4
LN = 16


@functools.lru_cache(maxsize=None)
def _build_sc_agg(n, npad, ng, d):
    assert npad % 128 == 0 and ng % SB == 0 and (ng // SB) % 2 == 0
    rpt = npad // NS
    assert rpt % 128 == 0
    drows = rpt // 128
    mesh = plsc.VectorSubcoreMesh(core_axis_name="c", subcore_axis_name="s")

    @functools.partial(
        pl.kernel,
        mesh=mesh,
        compiler_params=pltpu.CompilerParams(use_tc_tiling_on_sc=False,
                                             needs_layout_passes=False),
        out_type=(
            jax.ShapeDtypeStruct((NC, npad, d), jnp.float32),
            jax.ShapeDtypeStruct((NC, npad // 128, 128), jnp.float32),
        ),
        scratch_types=[
            pltpu.VMEM((2, SB, GB), jnp.int32),
            pltpu.VMEM((2, SB, GB), jnp.int32),
            pltpu.VMEM((NBUF, GB, d), jnp.float32),
            pltpu.VMEM((GB, LN), jnp.float32),
            pltpu.VMEM((80, LN), jnp.float32),
            pltpu.VMEM((drows, 128), jnp.float32),
            pltpu.VMEM_SHARED((npad, d), jnp.float32),
            pltpu.VMEM_SHARED((npad, LN), jnp.float32),
            pltpu.SemaphoreType.DMA((NBUF,)),
            pltpu.SemaphoreType.DMA((NBUF,)),
            pltpu.SemaphoreType.DMA,
            pltpu.SemaphoreType.DMA((2,)),
        ],
    )
    def sc_agg(feat_hbm, src_hbm, dst_hbm, z128_hbm, z16_hbm, ones_hbm,
               out_hbm, deg_hbm,
               sidx, didx, rows, ones, dbuf, ddense, acc, dacc,
               gsem, ssem, qsem, isem):
        c = lax.axis_index("c")
        s = lax.axis_index("s")
        wid = s * NC + c
        base_g = wid * ng

        def idx_load(sb, buf):
            pltpu.async_copy(src_hbm.at[pl.ds(base_g + sb * SB, SB)],
                             sidx.at[buf], isem.at[buf])
            pltpu.async_copy(dst_hbm.at[pl.ds(base_g + sb * SB, SB)],
                             didx.at[buf], isem.at[buf])

        def idx_wait(buf):
            pltpu.make_async_copy(src_hbm.at[pl.ds(base_g, SB)],
                                  sidx.at[buf], isem.at[buf]).wait()
            pltpu.make_async_copy(dst_hbm.at[pl.ds(base_g, SB)],
                                  didx.at[buf], isem.at[buf]).wait()

        idx_load(0, 0)
        pltpu.sync_copy(ones_hbm, ones)
        pltpu.sync_copy(z128_hbm.at[pl.ds(s * rpt, rpt)],
                        acc.at[pl.ds(s * rpt, rpt)])
        pltpu.sync_copy(z16_hbm.at[pl.ds(s * rpt, rpt)],
                        dacc.at[pl.ds(s * rpt, rpt)])
        idx_wait(0)
        for b in range(NBUF):
            pltpu.async_copy(feat_hbm.at[sidx.at[0, b]], rows.at[b],
                             gsem.at[b])
        idx_load(1, 1)
        plsc.subcore_barrier()

        def outer(sb, carry):
            ib = lax.rem(sb, 2)
            for k in range(SB):
                b = k % NBUF
                pltpu.make_async_copy(feat_hbm.at[sidx.at[ib, k]],
                                      rows.at[b], gsem.at[b]).wait()
                pltpu.async_copy(rows.at[b], acc.at[didx.at[ib, k]],
                                 ssem.at[b], add=True)
                pltpu.async_copy(ones, dacc.at[didx.at[ib, k]],
                                 qsem, add=True)
                if k < SB - NBUF:
                    pltpu.make_async_copy(rows.at[b], acc.at[didx.at[ib, k]],
                                          ssem.at[b]).wait()
                    pltpu.async_copy(feat_hbm.at[sidx.at[ib, k + NBUF]],
                                     rows.at[b], gsem.at[b])
                else:
                    kk = k - (SB - NBUF)

                    @pl.when(sb + 1 < ng // SB)
                    def _():
                        if kk == 0:
                            idx_wait(1 - ib)
                        pltpu.make_async_copy(rows.at[b],
                                              acc.at[didx.at[ib, k]],
                                              ssem.at[b]).wait()
                        pltpu.async_copy(feat_hbm.at[sidx.at[1 - ib, kk]],
                                         rows.at[b], gsem.at[b])

            for k in range(SB):
                pltpu.make_async_copy(ones, dacc.at[didx.at[0, 0]],
                                      qsem).wait()

            @pl.when(sb + 2 < ng // SB)
            def _():
                idx_load(sb + 2, ib)

            return carry

        lax.fori_loop(0, ng // SB, outer, 0)
        for b in range(NBUF):
            pltpu.make_async_copy(rows.at[b], acc.at[didx.at[0, b]],
                                  ssem.at[b]).wait()
        plsc.subcore_barrier()
        pltpu.sync_copy(acc.at[pl.ds(s * rpt, rpt)],
                        out_hbm.at[c, pl.ds(s * rpt, rpt)])
        col0 = jnp.zeros((LN,), jnp.int32)
        lane = lax.iota(jnp.int32, LN)
        for t in range(rpt // 80):
            pltpu.sync_copy(dacc.at[pl.ds(s * rpt + t * 80, 80)], dbuf)
            for j in range(5):
                vals = plsc.load_gather(dbuf, [lane + LN * j, col0])
                m = 5 * t + j
                ddense[m // 8, pl.ds((m % 8) * LN, LN)] = vals
        pltpu.sync_copy(ddense, deg_hbm.at[c, pl.ds(s * drows, drows)])

    return sc_agg


def _tc_body(p_ref, d_ref, feat_ref, w_ref, b_ref, out_ref):
    acc = p_ref[0] + p_ref[1]
    deg = d_ref[0] + d_ref[1]
    h = (acc + feat_ref[...]) / (deg + 1.0)
    r = lax.dot_general(h, w_ref[...], (((1,), (1,)), ((), ())),
                        preferred_element_type=jnp.float32)
    r = r + b_ref[...]
    out_ref[...] = jnp.where(r >= 0, r, 0.01 * r)


def kernel(feat, edge_index, W_neigh, b_neigh):
    n, d = feat.shape
    e = edge_index.shape[1]
    npad = -(-n // (NS * 128)) * (NS * 128)
    ng = -(-(-(-e // GB) // (NC * NS)) // (2 * SB)) * (2 * SB)
    e_pad = ng * NC * NS * GB
    pad_i = jnp.arange(e_pad - e, dtype=jnp.int32)
    src = jnp.concatenate([edge_index[0], pad_i % n])
    dst = jnp.concatenate([edge_index[1], n + pad_i % (npad - n)])
    src2d = src.reshape(e_pad // GB, GB)
    dst2d = dst.reshape(e_pad // GB, GB)
    z128 = jnp.zeros((npad, d), jnp.float32)
    z16 = jnp.zeros((npad, LN), jnp.float32)
    onehot = (lax.iota(jnp.int32, LN) == 0).astype(jnp.float32)
    ones_mat = jnp.tile(onehot[None, :], (GB, 1))

    partials, degp = _build_sc_agg(n, npad, ng, d)(
        feat, src2d, dst2d, z128, z16, ones_mat)
    deg3 = degp.reshape(NC, npad, 1)

    bn = 1024
    grid = npad // bn
    out = pl.pallas_call(
        _tc_body,
        grid=(grid,),
        in_specs=[
            pl.BlockSpec((NC, bn, d), lambda i: (0, i, 0)),
            pl.BlockSpec((NC, bn, 1), lambda i: (0, i, 0)),
            pl.BlockSpec((bn, d), lambda i: (i, 0)),
            pl.BlockSpec(W_neigh.shape, lambda i: (0, 0)),
            pl.BlockSpec((1, b_neigh.shape[0]), lambda i: (0, 0)),
        ],
        out_specs=pl.BlockSpec((bn, d), lambda i: (i, 0)),
        out_shape=jax.ShapeDtypeStruct((n, d), jnp.float32),
    )(partials, deg3, feat, W_neigh, b_neigh.reshape(1, -1))
    return out


# speedup vs baseline: 1.0073x; 1.0073x over previous
import functools

import jax
import jax.numpy as jnp
from jax import lax
from jax.experimental import pallas as pl
from jax.experimental.pallas import tpu as pltpu
from jax.experimental.pallas import tpu_sc as plsc

NC = 2
NS = 16
GB = 80
NBUF = 3
SB = 3
LN = 16


@functools.lru_cache(maxsize=None)
def _build_sc_agg(n, npad, ng, d):
    assert npad % 128 == 0 and ng % SB == 0 and (ng // SB) % 2 == 0
    rpt = npad // NS
    assert rpt % 128 == 0
    drows = rpt // 128
    mesh = plsc.VectorSubcoreMesh(core_axis_name="c", subcore_axis_name="s")

    @functools.partial(
        pl.kernel,
        mesh=mesh,
        compiler_params=pltpu.CompilerParams(use_tc_tiling_on_sc=False,
                                             needs_layout_passes=False),
        out_type=(
            jax.ShapeDtypeStruct((NC, npad, d), jnp.float32),
            jax.ShapeDtypeStruct((NC, npad // 128, 128), jnp.float32),
        ),
        scratch_types=[
            pltpu.VMEM((2, SB, GB), jnp.int32),
            pltpu.VMEM((2, SB, GB), jnp.int32),
            pltpu.VMEM((NBUF, GB, d), jnp.float32),
            pltpu.VMEM((GB, LN), jnp.float32),
            pltpu.VMEM((80, LN), jnp.float32),
            pltpu.VMEM((drows, 128), jnp.float32),
            pltpu.VMEM_SHARED((npad, d), jnp.float32),
            pltpu.VMEM_SHARED((npad, LN), jnp.float32),
            pltpu.SemaphoreType.DMA((NBUF,)),
            pltpu.SemaphoreType.DMA((NBUF,)),
            pltpu.SemaphoreType.DMA,
            pltpu.SemaphoreType.DMA((2,)),
        ],
    )
    def sc_agg(feat_hbm, src_hbm, dst_hbm, z128_hbm, z16_hbm, ones_hbm,
               out_hbm, deg_hbm,
               sidx, didx, rows, ones, dbuf, ddense, acc, dacc,
               gsem, ssem, qsem, isem):
        c = lax.axis_index("c")
        s = lax.axis_index("s")
        wid = s * NC + c
        base_g = wid * ng

        def idx_load(sb, buf):
            pltpu.async_copy(src_hbm.at[pl.ds(base_g + sb * SB, SB)],
                             sidx.at[buf], isem.at[buf])
            pltpu.async_copy(dst_hbm.at[pl.ds(base_g + sb * SB, SB)],
                             didx.at[buf], isem.at[buf])

        def idx_wait(buf):
            pltpu.make_async_copy(src_hbm.at[pl.ds(base_g, SB)],
                                  sidx.at[buf], isem.at[buf]).wait()
            pltpu.make_async_copy(dst_hbm.at[pl.ds(base_g, SB)],
                                  didx.at[buf], isem.at[buf]).wait()

        idx_load(0, 0)
        pltpu.sync_copy(ones_hbm, ones)
        pltpu.sync_copy(z128_hbm.at[pl.ds(s * rpt, rpt)],
                        acc.at[pl.ds(s * rpt, rpt)])
        pltpu.sync_copy(z16_hbm.at[pl.ds(s * rpt, rpt)],
                        dacc.at[pl.ds(s * rpt, rpt)])
        idx_wait(0)
        for b in range(NBUF):
            pltpu.async_copy(feat_hbm.at[sidx.at[0, b]], rows.at[b],
                             gsem.at[b])
        idx_load(1, 1)
        plsc.subcore_barrier()

        def outer(sb, carry):
            ib = lax.rem(sb, 2)
            for k in range(SB):
                b = k % NBUF
                pltpu.make_async_copy(feat_hbm.at[sidx.at[ib, k]],
                                      rows.at[b], gsem.at[b]).wait()
                pltpu.async_copy(rows.at[b], acc.at[didx.at[ib, k]],
                                 ssem.at[b], add=True)
                pltpu.async_copy(ones, dacc.at[didx.at[ib, k]],
                                 qsem, add=True)
                if k < SB - NBUF:
                    pltpu.make_async_copy(rows.at[b], acc.at[didx.at[ib, k]],
                                          ssem.at[b]).wait()
                    pltpu.async_copy(feat_hbm.at[sidx.at[ib, k + NBUF]],
                                     rows.at[b], gsem.at[b])
                else:
                    kk = k - (SB - NBUF)

                    @pl.when(sb + 1 < ng // SB)
                    def _():
                        if kk == 0:
                            idx_wait(1 - ib)
                        pltpu.make_async_copy(rows.at[b],
                                              acc.at[didx.at[ib, k]],
                                              ssem.at[b]).wait()
                        pltpu.async_copy(feat_hbm.at[sidx.at[1 - ib, kk]],
                                         rows.at[b], gsem.at[b])

            for k in range(SB):
                pltpu.make_async_copy(ones, dacc.at[didx.at[0, 0]],
                                      qsem).wait()

            @pl.when(sb + 2 < ng // SB)
            def _():
                idx_load(sb + 2, ib)

            return carry

        lax.fori_loop(0, ng // SB, outer, 0)
        for b in range(NBUF):
            pltpu.make_async_copy(rows.at[b], acc.at[didx.at[0, b]],
                                  ssem.at[b]).wait()
        plsc.subcore_barrier()
        pltpu.sync_copy(acc.at[pl.ds(s * rpt, rpt)],
                        out_hbm.at[c, pl.ds(s * rpt, rpt)])
        col0 = jnp.zeros((LN,), jnp.int32)
        lane = lax.iota(jnp.int32, LN)
        for t in range(rpt // 80):
            pltpu.sync_copy(dacc.at[pl.ds(s * rpt + t * 80, 80)], dbuf)
            for j in range(5):
                vals = plsc.load_gather(dbuf, [lane + LN * j, col0])
                m = 5 * t + j
                ddense[m // 8, pl.ds((m % 8) * LN, LN)] = vals
        pltpu.sync_copy(ddense, deg_hbm.at[c, pl.ds(s * drows, drows)])

    return sc_agg


def _tc_body(p_ref, d_ref, feat_ref, w_ref, b_ref, out_ref):
    acc = p_ref[0] + p_ref[1]
    deg = d_ref[0] + d_ref[1]
    h = (acc + feat_ref[...]) / (deg + 1.0)
    r = lax.dot_general(h, w_ref[...], (((1,), (1,)), ((), ())),
                        preferred_element_type=jnp.float32)
    r = r + b_ref[...]
    out_ref[...] = jnp.where(r >= 0, r, 0.01 * r)


def kernel(feat, edge_index, W_neigh, b_neigh):
    n, d = feat.shape
    e = edge_index.shape[1]
    npad = -(-n // (NS * 128)) * (NS * 128)
    ng = -(-(-(-e // GB) // (NC * NS)) // (2 * SB)) * (2 * SB)
    e_pad = ng * NC * NS * GB
    pad_i = jnp.arange(e_pad - e, dtype=jnp.int32)
    src = jnp.concatenate([edge_index[0], pad_i % n])
    dst = jnp.concatenate([edge_index[1], n + pad_i % (npad - n)])
    src2d = src.reshape(e_pad // GB, GB)
    dst2d = dst.reshape(e_pad // GB, GB)
    z128 = jnp.zeros((npad, d), jnp.float32)
    z16 = jnp.zeros((npad, LN), jnp.float32)
    onehot = (lax.iota(jnp.int32, LN) == 0).astype(jnp.float32)
    ones_mat = jnp.tile(onehot[None, :], (GB, 1))

    partials, degp = _build_sc_agg(n, npad, ng, d)(
        feat, src2d, dst2d, z128, z16, ones_mat)
    deg3 = degp.reshape(NC, npad, 1)

    bn = 1024
    grid = npad // bn
    out = pl.pallas_call(
        _tc_body,
        grid=(grid,),
        in_specs=[
            pl.BlockSpec((NC, bn, d), lambda i: (0, i, 0)),
            pl.BlockSpec((NC, bn, 1), lambda i: (0, i, 0)),
            pl.BlockSpec((bn, d), lambda i: (i, 0)),
            pl.BlockSpec(W_neigh.shape, lambda i: (0, 0)),
            pl.BlockSpec((1, b_neigh.shape[0]), lambda i: (0, 0)),
        ],
        out_specs=pl.BlockSpec((bn, d), lambda i: (i, 0)),
        out_shape=jax.ShapeDtypeStruct((n, d), jnp.float32),
    )(partials, deg3, feat, W_neigh, b_neigh.reshape(1, -1))
    return out
